# Initial kernel scaffold; baseline (speedup 1.0000x reference)
import jax
import jax.numpy as jnp
from jax.experimental import pallas as pl


def kernel(x, edge_index, edge_weight, batch, W1, b1, W2, b2, W3, b3, Wf, bf):
    raise NotImplementedError("write your pallas kernel here")


# speedup vs baseline: 5.4150x; 5.4150x over previous
import functools

import jax
import jax.numpy as jnp
from jax import lax
from jax.experimental import pallas as pl
from jax.experimental.pallas import tpu as pltpu
from jax.experimental.pallas import tpu_sc as plsc

NC = 2
NS = 16
NW = NC * NS
DEGW = 16
CAP = 16384
CSZ = 800
GC = 128

_SC_PARAMS = pltpu.CompilerParams(needs_layout_passes=False)
_MESH = dict(core_axis_name="c", subcore_axis_name="s")


def _wid():
    return lax.axis_index("c") * NS + lax.axis_index("s")


def _part_body(n_pad, e, dst_hbm, src_hbm, ew_hbm,
               msrc_hbm, mdl_hbm, mew_hbm, cnt_hbm, deg_hbm,
               dstv, srcv, ewv, msrcv, mdlv, mewv, cntv, dega):
    w = _wid()
    stripe = n_pad // NW
    lo = w * stripe
    z16i = jnp.zeros((16,), jnp.int32)
    z16f = jnp.zeros((16,), jnp.float32)

    def zlist(i, c):
        msrcv[pl.ds(16 * i, 16)] = z16i
        mdlv[pl.ds(16 * i, 16)] = z16i
        mewv[pl.ds(16 * i, 16)] = z16f
        return c

    lax.fori_loop(0, CAP // 16, zlist, 0)

    def zdeg(i, c):
        dega[i] = z16f
        return c

    lax.fori_loop(0, stripe, zdeg, 0)

    def chunk(ci, off):
        coff = ci * CSZ
        pltpu.sync_copy(dst_hbm.at[pl.ds(coff, CSZ)], dstv)
        pltpu.sync_copy(src_hbm.at[pl.ds(coff, CSZ)], srcv)
        pltpu.sync_copy(ew_hbm.at[pl.ds(coff, CSZ)], ewv)

        def vec(i, off2):
            dl = dstv[pl.ds(16 * i, 16)] - lo
            m = (dl >= 0) & (dl < stripe)
            sv = srcv[pl.ds(16 * i, 16)]
            wv = jnp.abs(ewv[pl.ds(16 * i, 16)])
            o = jnp.minimum(off2, CAP - 16)
            plsc.store_compressed(mdlv.at[pl.ds(o, 16)], dl, mask=m)
            plsc.store_compressed(msrcv.at[pl.ds(o, 16)], sv, mask=m)
            plsc.store_compressed(mewv.at[pl.ds(o, 16)], wv, mask=m)
            npop = plsc.all_reduce_population_count(m)
            return off2 + jnp.max(npop)

        return lax.fori_loop(0, CSZ // 16, vec, off)

    off = lax.fori_loop(0, e // CSZ, chunk, jnp.int32(0))

    cols = lax.iota(jnp.int32, 16)

    def dedge(r, c):
        gi = jnp.full((16,), r, jnp.int32)
        dlv = plsc.load_gather(mdlv, [gi])
        wv = plsc.load_gather(mewv, [gi])
        plsc.addupdate_scatter(dega, [dlv, cols], wv)
        return c

    lax.fori_loop(0, off, dedge, 0)

    pltpu.sync_copy(msrcv, msrc_hbm.at[pl.ds(w * CAP, CAP)])
    pltpu.sync_copy(mdlv, mdl_hbm.at[pl.ds(w * CAP, CAP)])
    pltpu.sync_copy(mewv, mew_hbm.at[pl.ds(w * CAP, CAP)])
    cntv[...] = jnp.full((16,), off, jnp.int32)
    pltpu.sync_copy(cntv, cnt_hbm.at[pl.ds(w * 16, 16)])
    pltpu.sync_copy(dega, deg_hbm.at[pl.ds(lo, stripe)])


def _layer_body(n_pad, fg, fs, xw_hbm, msrc_hbm, mdl_hbm, mew_hbm, cnt_hbm,
                out_hbm, msrcv, mdlv, mewv, cntv, rows, accum, sem):
    w = _wid()
    stripe = n_pad // NW
    lo = w * stripe
    nf = fs // 16
    pltpu.sync_copy(msrc_hbm.at[pl.ds(w * CAP, CAP)], msrcv)
    pltpu.sync_copy(mdl_hbm.at[pl.ds(w * CAP, CAP)], mdlv)
    pltpu.sync_copy(mew_hbm.at[pl.ds(w * CAP, CAP)], mewv)
    pltpu.sync_copy(cnt_hbm.at[pl.ds(w * 16, 16)], cntv)
    z16f = jnp.zeros((16,), jnp.float32)

    def zacc(i, c):
        for j in range(nf):
            accum[i, pl.ds(16 * j, 16)] = z16f
        return c

    lax.fori_loop(0, stripe, zacc, 0)

    cnt = jnp.max(cntv[...])
    trips = (cnt + GC - 1) // GC
    cols = [lax.iota(jnp.int32, 16) + 16 * j for j in range(nf)]

    def trip(t, c):
        pltpu.async_copy(xw_hbm.at[msrcv.at[pl.ds(t * GC, GC)]], rows,
                         sem).wait()
        base = t * GC

        def edge(r, c2):
            gi = jnp.full((16,), base + r, jnp.int32)
            dlv = plsc.load_gather(mdlv, [gi])
            wv = plsc.load_gather(mewv, [gi])
            for j in range(nf):
                vals = rows[r, pl.ds(16 * j, 16)] * wv
                plsc.addupdate_scatter(accum, [dlv, cols[j]], vals)
            return c2

        lax.fori_loop(0, GC, edge, 0)
        return c

    lax.fori_loop(0, trips, trip, 0)
    pltpu.sync_copy(accum, out_hbm.at[pl.ds(lo, stripe)])


def _make_part(n_pad, e):
    mesh = plsc.VectorSubcoreMesh(**_MESH)
    return functools.partial(
        pl.kernel,
        mesh=mesh,
        out_type=(
            jax.ShapeDtypeStruct((NW * CAP,), jnp.int32),
            jax.ShapeDtypeStruct((NW * CAP,), jnp.int32),
            jax.ShapeDtypeStruct((NW * CAP,), jnp.float32),
            jax.ShapeDtypeStruct((NW * 16,), jnp.int32),
            jax.ShapeDtypeStruct((n_pad, DEGW), jnp.float32),
        ),
        scratch_types=[
            pltpu.VMEM((CSZ,), jnp.int32),
            pltpu.VMEM((CSZ,), jnp.int32),
            pltpu.VMEM((CSZ,), jnp.float32),
            pltpu.VMEM((CAP,), jnp.int32),
            pltpu.VMEM((CAP,), jnp.int32),
            pltpu.VMEM((CAP,), jnp.float32),
            pltpu.VMEM((16,), jnp.int32),
            pltpu.VMEM((n_pad // NW, DEGW), jnp.float32),
        ],
        compiler_params=_SC_PARAMS,
    )(functools.partial(_part_body, n_pad, e))


def _make_layer(n_pad, fg, fs):
    mesh = plsc.VectorSubcoreMesh(**_MESH)
    return functools.partial(
        pl.kernel,
        mesh=mesh,
        out_type=jax.ShapeDtypeStruct((n_pad, fs), jnp.float32),
        scratch_types=[
            pltpu.VMEM((CAP,), jnp.int32),
            pltpu.VMEM((CAP,), jnp.int32),
            pltpu.VMEM((CAP,), jnp.float32),
            pltpu.VMEM((16,), jnp.int32),
            pltpu.VMEM((GC, fg), jnp.float32),
            pltpu.VMEM((n_pad // NW, fs), jnp.float32),
            pltpu.SemaphoreType.DMA,
        ],
        compiler_params=_SC_PARAMS,
    )(functools.partial(_layer_body, n_pad, fg, fs))


def _prep_body(n, fp, x_ref, w1_ref, deg_ref, pre_ref, dinv_ref):
    deg = deg_ref[:n, 0:1] + 1.0
    dinv = lax.rsqrt(deg)
    dinvf = jnp.broadcast_to(dinv, (n, fp))
    pre = jnp.dot(x_ref[...], w1_ref[...], preferred_element_type=jnp.float32)
    pre_ref[...] = pre * dinvf
    dinv_ref[...] = dinvf


def _combine_body(n, fs, s_ref, pre_ref, dinv_ref, b_ref, w_ref, out_ref):
    pre = pre_ref[...]
    s = pre[:, :fs] + s_ref[:n]
    s = jnp.concatenate([s, jnp.zeros_like(pre[:, fs:])], axis=1)
    h = jnp.maximum(s * dinv_ref[...] + b_ref[...], 0.0)
    out_ref[...] = (jnp.dot(h, w_ref[...], preferred_element_type=jnp.float32)
                    * dinv_ref[...])


def _final_body(n, g, f3p, s_ref, pre_ref, dinv_ref, b_ref, batch_ref,
                wf_ref, bf_ref, out_ref):
    h = jnp.maximum(
        (s_ref[:n] + pre_ref[:, :f3p]) * dinv_ref[:, :f3p] + b_ref[...], 0.0)
    onehot = jnp.equal(
        batch_ref[...],
        lax.broadcasted_iota(jnp.int32, (n, g), 1)).astype(jnp.float32)
    h_aug = jnp.concatenate([h, jnp.ones((n, 1), jnp.float32)], axis=1)
    pooled_aug = lax.dot_general(onehot, h_aug, (((0,), (0,)), ((), ())),
                                 preferred_element_type=jnp.float32)
    cnt = pooled_aug[:, -1:]
    pooled = pooled_aug[:, :-1] / jnp.maximum(cnt, 1.0)
    logits = (jnp.dot(pooled, wf_ref[...], preferred_element_type=jnp.float32)
              + bf_ref[...])
    out_ref[...] = jax.nn.softmax(logits, axis=1)


def _tc_call(body, out_shapes):
    return pl.pallas_call(body, out_shape=out_shapes)


def kernel(x, edge_index, edge_weight, batch, W1, b1, W2, b2, W3, b3, Wf, bf):
    n, din = x.shape
    e = edge_index.shape[1]
    f1 = W1.shape[1]
    fp = 128
    f3p = 32
    g = 64
    out_dim = Wf.shape[1]
    n_pad = ((n + 8 * NW - 1) // (8 * NW)) * 8 * NW

    src = edge_index[0]
    dst = edge_index[1]
    w1p = jnp.pad(W1, ((0, 0), (0, fp - f1)))
    w2p = jnp.pad(W2, ((0, fp - f1), (0, fp - f1)))
    w3p = jnp.pad(W3, ((0, fp - f1), (0, fp - W3.shape[1])))
    b1p = jnp.pad(b1, (0, fp - f1)).reshape(1, fp)
    b2p = jnp.pad(b2, (0, fp - f1)).reshape(1, fp)
    b3p = jnp.pad(b3, (0, f3p - b3.shape[0])).reshape(1, f3p)
    wfp = jnp.pad(Wf, ((0, f3p - Wf.shape[0]), (0, 0)))
    fdt = jnp.float32

    msrc, mdl, mew, cnts, deg = _make_part(n_pad, e)(dst, src, edge_weight)
    pre1, dinv = _tc_call(
        functools.partial(_prep_body, n, fp),
        (jax.ShapeDtypeStruct((n, fp), fdt),
         jax.ShapeDtypeStruct((n, fp), fdt)),
    )(x, w1p, deg)

    layer96 = _make_layer(n_pad, fp, f1)
    layer32 = _make_layer(n_pad, fp, f3p)

    s1 = layer96(pre1, msrc, mdl, mew, cnts)
    pre2 = _tc_call(
        functools.partial(_combine_body, n, f1),
        jax.ShapeDtypeStruct((n, fp), fdt),
    )(s1, pre1, dinv, b1p, w2p)

    s2 = layer96(pre2, msrc, mdl, mew, cnts)
    pre3 = _tc_call(
        functools.partial(_combine_body, n, f1),
        jax.ShapeDtypeStruct((n, fp), fdt),
    )(s2, pre2, dinv, b2p, w3p)

    s3 = layer32(pre3, msrc, mdl, mew, cnts)
    out = _tc_call(
        functools.partial(_final_body, n, g, f3p),
        jax.ShapeDtypeStruct((g, out_dim), fdt),
    )(s3, pre3, dinv, b3p, batch.reshape(n, 1), wfp,
      bf.reshape(1, out_dim))
    return out


# speedup vs baseline: 5.4889x; 1.0136x over previous
import functools

import jax
import jax.numpy as jnp
from jax import lax
from jax.experimental import pallas as pl
from jax.experimental.pallas import tpu as pltpu
from jax.experimental.pallas import tpu_sc as plsc

NC = 2
NS = 16
NW = NC * NS
DEGW = 16
CAP = 16384
CSZ = 800
GC = 128

_SC_PARAMS = pltpu.CompilerParams(needs_layout_passes=False)
_MESH = dict(core_axis_name="c", subcore_axis_name="s")


def _wid():
    return lax.axis_index("c") * NS + lax.axis_index("s")


def _part_body(n_pad, e, dst_hbm, src_hbm, ew_hbm,
               msrc_hbm, mdl_hbm, mew_hbm, cnt_hbm, deg_hbm,
               dstv, srcv, ewv, msrcv, mdlv, mewv, cntv, dega):
    w = _wid()
    stripe = n_pad // NW
    lo = w * stripe
    z16i = jnp.zeros((16,), jnp.int32)
    z16f = jnp.zeros((16,), jnp.float32)

    def zlist(i, c):
        msrcv[pl.ds(16 * i, 16)] = z16i
        mdlv[pl.ds(16 * i, 16)] = z16i
        mewv[pl.ds(16 * i, 16)] = z16f
        return c

    lax.fori_loop(0, CAP // 16, zlist, 0, unroll=4)

    def zdeg(i, c):
        dega[i] = z16f
        return c

    lax.fori_loop(0, stripe, zdeg, 0, unroll=4)

    def chunk(ci, off):
        coff = ci * CSZ
        pltpu.sync_copy(dst_hbm.at[pl.ds(coff, CSZ)], dstv)
        pltpu.sync_copy(src_hbm.at[pl.ds(coff, CSZ)], srcv)
        pltpu.sync_copy(ew_hbm.at[pl.ds(coff, CSZ)], ewv)

        def vec(i, off2):
            dl = dstv[pl.ds(16 * i, 16)] - lo
            m = (dl >= 0) & (dl < stripe)
            sv = srcv[pl.ds(16 * i, 16)]
            wv = jnp.abs(ewv[pl.ds(16 * i, 16)])
            o = jnp.minimum(off2, CAP - 16)
            plsc.store_compressed(mdlv.at[pl.ds(o, 16)], dl, mask=m)
            plsc.store_compressed(msrcv.at[pl.ds(o, 16)], sv, mask=m)
            plsc.store_compressed(mewv.at[pl.ds(o, 16)], wv, mask=m)
            npop = plsc.all_reduce_population_count(m)
            return off2 + lax.squeeze(lax.slice_in_dim(npop, 0, 1, axis=0), [0])

        return lax.fori_loop(0, CSZ // 16, vec, off, unroll=4)

    off = lax.fori_loop(0, e // CSZ, chunk, jnp.int32(0))

    cols = lax.iota(jnp.int32, 16)

    def dedge(g2, c):
        for k in range(4):
            gi = jnp.full((16,), 4 * g2 + k, jnp.int32)
            dlv = plsc.load_gather(mdlv, [gi])
            wv = plsc.load_gather(mewv, [gi])
            plsc.addupdate_scatter(dega, [dlv, cols], wv)
        return c

    lax.fori_loop(0, (off + 3) // 4, dedge, 0)

    pltpu.sync_copy(msrcv, msrc_hbm.at[pl.ds(w * CAP, CAP)])
    pltpu.sync_copy(mdlv, mdl_hbm.at[pl.ds(w * CAP, CAP)])
    pltpu.sync_copy(mewv, mew_hbm.at[pl.ds(w * CAP, CAP)])
    cntv[...] = jnp.full((16,), off, jnp.int32)
    pltpu.sync_copy(cntv, cnt_hbm.at[pl.ds(w * 16, 16)])
    pltpu.sync_copy(dega, deg_hbm.at[pl.ds(lo, stripe)])


def _layer_body(n_pad, fg, fs, xw_hbm, msrc_hbm, mdl_hbm, mew_hbm, cnt_hbm,
                out_hbm, msrcv, mdlv, mewv, cntv, rows, accum, sem):
    w = _wid()
    stripe = n_pad // NW
    lo = w * stripe
    nf = fs // 16
    pltpu.sync_copy(msrc_hbm.at[pl.ds(w * CAP, CAP)], msrcv)
    pltpu.sync_copy(mdl_hbm.at[pl.ds(w * CAP, CAP)], mdlv)
    pltpu.sync_copy(mew_hbm.at[pl.ds(w * CAP, CAP)], mewv)
    pltpu.sync_copy(cnt_hbm.at[pl.ds(w * 16, 16)], cntv)
    z16f = jnp.zeros((16,), jnp.float32)

    def zacc(i, c):
        for j in range(nf):
            accum[i, pl.ds(16 * j, 16)] = z16f
        return c

    lax.fori_loop(0, stripe, zacc, 0, unroll=4)

    cnt = jnp.max(cntv[...])
    trips = (cnt + GC - 1) // GC
    cols = [lax.iota(jnp.int32, 16) + 16 * j for j in range(nf)]

    def trip(t, c):
        pltpu.async_copy(xw_hbm.at[msrcv.at[pl.ds(t * GC, GC)]], rows,
                         sem).wait()
        base = t * GC

        def edge(r, c2):
            gi = jnp.full((16,), base + r, jnp.int32)
            dlv = plsc.load_gather(mdlv, [gi])
            wv = plsc.load_gather(mewv, [gi])
            for j in range(nf):
                vals = rows[r, pl.ds(16 * j, 16)] * wv
                plsc.addupdate_scatter(accum, [dlv, cols[j]], vals)
            return c2

        lax.fori_loop(0, GC, edge, 0, unroll=4)
        return c

    lax.fori_loop(0, trips, trip, 0)
    pltpu.sync_copy(accum, out_hbm.at[pl.ds(lo, stripe)])


def _make_part(n_pad, e):
    mesh = plsc.VectorSubcoreMesh(**_MESH)
    return functools.partial(
        pl.kernel,
        mesh=mesh,
        out_type=(
            jax.ShapeDtypeStruct((NW * CAP,), jnp.int32),
            jax.ShapeDtypeStruct((NW * CAP,), jnp.int32),
            jax.ShapeDtypeStruct((NW * CAP,), jnp.float32),
            jax.ShapeDtypeStruct((NW * 16,), jnp.int32),
            jax.ShapeDtypeStruct((n_pad, DEGW), jnp.float32),
        ),
        scratch_types=[
            pltpu.VMEM((CSZ,), jnp.int32),
            pltpu.VMEM((CSZ,), jnp.int32),
            pltpu.VMEM((CSZ,), jnp.float32),
            pltpu.VMEM((CAP,), jnp.int32),
            pltpu.VMEM((CAP,), jnp.int32),
            pltpu.VMEM((CAP,), jnp.float32),
            pltpu.VMEM((16,), jnp.int32),
            pltpu.VMEM((n_pad // NW, DEGW), jnp.float32),
        ],
        compiler_params=_SC_PARAMS,
    )(functools.partial(_part_body, n_pad, e))


def _make_layer(n_pad, fg, fs):
    mesh = plsc.VectorSubcoreMesh(**_MESH)
    return functools.partial(
        pl.kernel,
        mesh=mesh,
        out_type=jax.ShapeDtypeStruct((n_pad, fs), jnp.float32),
        scratch_types=[
            pltpu.VMEM((CAP,), jnp.int32),
            pltpu.VMEM((CAP,), jnp.int32),
            pltpu.VMEM((CAP,), jnp.float32),
            pltpu.VMEM((16,), jnp.int32),
            pltpu.VMEM((GC, fg), jnp.float32),
            pltpu.VMEM((n_pad // NW, fs), jnp.float32),
            pltpu.SemaphoreType.DMA,
        ],
        compiler_params=_SC_PARAMS,
    )(functools.partial(_layer_body, n_pad, fg, fs))


def _prep_body(n, fp, x_ref, w1_ref, deg_ref, pre_ref, dinv_ref):
    deg = deg_ref[:n, 0:1] + 1.0
    dinv = lax.rsqrt(deg)
    dinvf = jnp.broadcast_to(dinv, (n, fp))
    pre = jnp.dot(x_ref[...], w1_ref[...], preferred_element_type=jnp.float32)
    pre_ref[...] = pre * dinvf
    dinv_ref[...] = dinvf


def _combine_body(n, fs, s_ref, pre_ref, dinv_ref, b_ref, w_ref, out_ref):
    pre = pre_ref[...]
    s = pre[:, :fs] + s_ref[:n]
    s = jnp.concatenate([s, jnp.zeros_like(pre[:, fs:])], axis=1)
    h = jnp.maximum(s * dinv_ref[...] + b_ref[...], 0.0)
    out_ref[...] = (jnp.dot(h, w_ref[...], preferred_element_type=jnp.float32)
                    * dinv_ref[...])


def _final_body(n, g, f3p, s_ref, pre_ref, dinv_ref, b_ref, batch_ref,
                wf_ref, bf_ref, out_ref):
    h = jnp.maximum(
        (s_ref[:n] + pre_ref[:, :f3p]) * dinv_ref[:, :f3p] + b_ref[...], 0.0)
    onehot = jnp.equal(
        batch_ref[...],
        lax.broadcasted_iota(jnp.int32, (n, g), 1)).astype(jnp.float32)
    h_aug = jnp.concatenate([h, jnp.ones((n, 1), jnp.float32)], axis=1)
    pooled_aug = lax.dot_general(onehot, h_aug, (((0,), (0,)), ((), ())),
                                 preferred_element_type=jnp.float32)
    cnt = pooled_aug[:, -1:]
    pooled = pooled_aug[:, :-1] / jnp.maximum(cnt, 1.0)
    logits = (jnp.dot(pooled, wf_ref[...], preferred_element_type=jnp.float32)
              + bf_ref[...])
    out_ref[...] = jax.nn.softmax(logits, axis=1)


def _tc_call(body, out_shapes):
    return pl.pallas_call(body, out_shape=out_shapes)


def kernel(x, edge_index, edge_weight, batch, W1, b1, W2, b2, W3, b3, Wf, bf):
    n, din = x.shape
    e = edge_index.shape[1]
    f1 = W1.shape[1]
    fp = 128
    f3p = 32
    g = 64
    out_dim = Wf.shape[1]
    n_pad = ((n + 8 * NW - 1) // (8 * NW)) * 8 * NW

    src = edge_index[0]
    dst = edge_index[1]
    w1p = jnp.pad(W1, ((0, 0), (0, fp - f1)))
    w2p = jnp.pad(W2, ((0, fp - f1), (0, fp - f1)))
    w3p = jnp.pad(W3, ((0, fp - f1), (0, fp - W3.shape[1])))
    b1p = jnp.pad(b1, (0, fp - f1)).reshape(1, fp)
    b2p = jnp.pad(b2, (0, fp - f1)).reshape(1, fp)
    b3p = jnp.pad(b3, (0, f3p - b3.shape[0])).reshape(1, f3p)
    wfp = jnp.pad(Wf, ((0, f3p - Wf.shape[0]), (0, 0)))
    fdt = jnp.float32

    msrc, mdl, mew, cnts, deg = _make_part(n_pad, e)(dst, src, edge_weight)
    pre1, dinv = _tc_call(
        functools.partial(_prep_body, n, fp),
        (jax.ShapeDtypeStruct((n, fp), fdt),
         jax.ShapeDtypeStruct((n, fp), fdt)),
    )(x, w1p, deg)

    layer96 = _make_layer(n_pad, fp, f1)
    layer32 = _make_layer(n_pad, fp, f3p)

    s1 = layer96(pre1, msrc, mdl, mew, cnts)
    pre2 = _tc_call(
        functools.partial(_combine_body, n, f1),
        jax.ShapeDtypeStruct((n, fp), fdt),
    )(s1, pre1, dinv, b1p, w2p)

    s2 = layer96(pre2, msrc, mdl, mew, cnts)
    pre3 = _tc_call(
        functools.partial(_combine_body, n, f1),
        jax.ShapeDtypeStruct((n, fp), fdt),
    )(s2, pre2, dinv, b2p, w3p)

    s3 = layer32(pre3, msrc, mdl, mew, cnts)
    out = _tc_call(
        functools.partial(_final_body, n, g, f3p),
        jax.ShapeDtypeStruct((g, out_dim), fdt),
    )(s3, pre3, dinv, b3p, batch.reshape(n, 1), wfp,
      bf.reshape(1, out_dim))
    return out


# speedup vs baseline: 9.0085x; 1.6412x over previous
import functools

import jax
import jax.numpy as jnp
from jax import lax
from jax.experimental import pallas as pl
from jax.experimental.pallas import tpu as pltpu
from jax.experimental.pallas import tpu_sc as plsc

NC = 2
NS = 16
NW = NC * NS
DEGW = 16
CAP = 16384
CSZ = 3200
GC = 128

_SC_PARAMS = pltpu.CompilerParams(needs_layout_passes=False)
_MESH = dict(core_axis_name="c", subcore_axis_name="s")


def _wid():
    return lax.axis_index("c") * NS + lax.axis_index("s")


def _part_body(n_pad, e, dst_hbm, src_hbm, ew_hbm,
               msrc_hbm, mdl_hbm, mew_hbm, cnt_hbm, deg_hbm,
               dstv, srcv, ewv, msrcv, mdlv, mewv, cntv, dega, sem_a, sem_b):
    w = _wid()
    stripe = n_pad // NW
    lo = w * stripe
    z16i = jnp.zeros((16,), jnp.int32)
    z16f = jnp.zeros((16,), jnp.float32)

    def zlist(i, c):
        msrcv[pl.ds(16 * i, 16)] = z16i
        mdlv[pl.ds(16 * i, 16)] = z16i
        mewv[pl.ds(16 * i, 16)] = z16f
        return c

    lax.fori_loop(0, CAP // 16, zlist, 0, unroll=4)

    def zdeg(i, c):
        dega[i] = z16f
        return c

    lax.fori_loop(0, stripe, zdeg, 0, unroll=4)

    nch = e // CSZ
    sems = (sem_a, sem_b)

    def _start(ci, p):
        coff = ci * CSZ
        pltpu.async_copy(dst_hbm.at[pl.ds(coff, CSZ)], dstv.at[p], sems[p])
        pltpu.async_copy(src_hbm.at[pl.ds(coff, CSZ)], srcv.at[p], sems[p])
        pltpu.async_copy(ew_hbm.at[pl.ds(coff, CSZ)], ewv.at[p], sems[p])

    def _wait(p):
        pltpu.make_async_copy(dst_hbm.at[pl.ds(0, CSZ)], dstv.at[p],
                              sems[p]).wait()
        pltpu.make_async_copy(src_hbm.at[pl.ds(0, CSZ)], srcv.at[p],
                              sems[p]).wait()
        pltpu.make_async_copy(ew_hbm.at[pl.ds(0, CSZ)], ewv.at[p],
                              sems[p]).wait()

    def _scan(p, off):
        def vec(i, off2):
            dl = dstv[p, pl.ds(16 * i, 16)] - lo
            m = (dl >= 0) & (dl < stripe)
            sv = srcv[p, pl.ds(16 * i, 16)]
            wv = jnp.abs(ewv[p, pl.ds(16 * i, 16)])
            o = jnp.minimum(off2, CAP - 16)
            plsc.store_compressed(mdlv.at[pl.ds(o, 16)], dl, mask=m)
            plsc.store_compressed(msrcv.at[pl.ds(o, 16)], sv, mask=m)
            plsc.store_compressed(mewv.at[pl.ds(o, 16)], wv, mask=m)
            npop = plsc.all_reduce_population_count(m)
            return off2 + lax.squeeze(lax.slice_in_dim(npop, 0, 1, axis=0), [0])

        return lax.fori_loop(0, CSZ // 16, vec, off, unroll=4)

    _start(0, 0)

    def pair(g2, off):
        ci0 = 2 * g2
        _start(ci0 + 1, 1)
        _wait(0)
        off = _scan(0, off)

        @pl.when(ci0 + 2 < nch)
        def _():
            _start(ci0 + 2, 0)

        _wait(1)
        off = _scan(1, off)
        return off

    off = lax.fori_loop(0, nch // 2, pair, jnp.int32(0))

    cols = lax.iota(jnp.int32, 16)

    def dedge(g2, c):
        for k in range(4):
            gi = jnp.full((16,), 4 * g2 + k, jnp.int32)
            dlv = plsc.load_gather(mdlv, [gi])
            wv = plsc.load_gather(mewv, [gi])
            plsc.addupdate_scatter(dega, [dlv, cols], wv)
        return c

    lax.fori_loop(0, (off + 3) // 4, dedge, 0)

    pltpu.sync_copy(msrcv, msrc_hbm.at[pl.ds(w * CAP, CAP)])
    pltpu.sync_copy(mdlv, mdl_hbm.at[pl.ds(w * CAP, CAP)])
    pltpu.sync_copy(mewv, mew_hbm.at[pl.ds(w * CAP, CAP)])
    cntv[...] = jnp.full((16,), off, jnp.int32)
    pltpu.sync_copy(cntv, cnt_hbm.at[pl.ds(w * 16, 16)])
    pltpu.sync_copy(dega, deg_hbm.at[pl.ds(lo, stripe)])


def _layer_body(n_pad, fg, fs, xw_hbm, msrc_hbm, mdl_hbm, mew_hbm, cnt_hbm,
                out_hbm, msrcv, mdlv, mewv, cntv, rows, accum, sem_a, sem_b):
    w = _wid()
    stripe = n_pad // NW
    lo = w * stripe
    nf = fs // 16
    pltpu.sync_copy(msrc_hbm.at[pl.ds(w * CAP, CAP)], msrcv)
    pltpu.sync_copy(mdl_hbm.at[pl.ds(w * CAP, CAP)], mdlv)
    pltpu.sync_copy(mew_hbm.at[pl.ds(w * CAP, CAP)], mewv)
    pltpu.sync_copy(cnt_hbm.at[pl.ds(w * 16, 16)], cntv)
    z16f = jnp.zeros((16,), jnp.float32)

    def zacc(i, c):
        for j in range(nf):
            accum[i, pl.ds(16 * j, 16)] = z16f
        return c

    lax.fori_loop(0, stripe, zacc, 0, unroll=4)

    cnt = jnp.max(cntv[...])
    trips = (cnt + GC - 1) // GC
    cols = [lax.iota(jnp.int32, 16) + 16 * j for j in range(nf)]
    sems = (sem_a, sem_b)

    def _start(t, p):
        pltpu.async_copy(xw_hbm.at[msrcv.at[pl.ds(t * GC, GC)]], rows.at[p],
                         sems[p])

    def _wait(p):
        pltpu.make_async_copy(xw_hbm.at[pl.ds(0, GC)], rows.at[p],
                              sems[p]).wait()

    def _process(t, p):
        base = t * GC

        def edge(r, c2):
            gi = jnp.full((16,), base + r, jnp.int32)
            dlv = plsc.load_gather(mdlv, [gi])
            wv = plsc.load_gather(mewv, [gi])
            for j in range(nf):
                vals = rows[p, r, pl.ds(16 * j, 16)] * wv
                plsc.addupdate_scatter(accum, [dlv, cols[j]], vals)
            return c2

        lax.fori_loop(0, GC, edge, 0, unroll=4)

    @pl.when(trips > 0)
    def _():
        _start(0, 0)

    def pair(g2, c):
        t0 = 2 * g2

        @pl.when(t0 + 1 < trips)
        def _():
            _start(t0 + 1, 1)

        _wait(0)
        _process(t0, 0)

        @pl.when(t0 + 2 < trips)
        def _():
            _start(t0 + 2, 0)

        @pl.when(t0 + 1 < trips)
        def _():
            _wait(1)
            _process(t0 + 1, 1)

        return c

    lax.fori_loop(0, (trips + 1) // 2, pair, 0)
    pltpu.sync_copy(accum, out_hbm.at[pl.ds(lo, stripe)])


def _make_part(n_pad, e):
    mesh = plsc.VectorSubcoreMesh(**_MESH)
    return functools.partial(
        pl.kernel,
        mesh=mesh,
        out_type=(
            jax.ShapeDtypeStruct((NW * CAP,), jnp.int32),
            jax.ShapeDtypeStruct((NW * CAP,), jnp.int32),
            jax.ShapeDtypeStruct((NW * CAP,), jnp.float32),
            jax.ShapeDtypeStruct((NW * 16,), jnp.int32),
            jax.ShapeDtypeStruct((n_pad, DEGW), jnp.float32),
        ),
        scratch_types=[
            pltpu.VMEM((2, CSZ), jnp.int32),
            pltpu.VMEM((2, CSZ), jnp.int32),
            pltpu.VMEM((2, CSZ), jnp.float32),
            pltpu.VMEM((CAP,), jnp.int32),
            pltpu.VMEM((CAP,), jnp.int32),
            pltpu.VMEM((CAP,), jnp.float32),
            pltpu.VMEM((16,), jnp.int32),
            pltpu.VMEM((n_pad // NW, DEGW), jnp.float32),
            pltpu.SemaphoreType.DMA,
            pltpu.SemaphoreType.DMA,
        ],
        compiler_params=_SC_PARAMS,
    )(functools.partial(_part_body, n_pad, e))


def _make_layer(n_pad, fg, fs):
    mesh = plsc.VectorSubcoreMesh(**_MESH)
    return functools.partial(
        pl.kernel,
        mesh=mesh,
        out_type=jax.ShapeDtypeStruct((n_pad, fs), jnp.float32),
        scratch_types=[
            pltpu.VMEM((CAP,), jnp.int32),
            pltpu.VMEM((CAP,), jnp.int32),
            pltpu.VMEM((CAP,), jnp.float32),
            pltpu.VMEM((16,), jnp.int32),
            pltpu.VMEM((2, GC, fg), jnp.float32),
            pltpu.VMEM((n_pad // NW, fs), jnp.float32),
            pltpu.SemaphoreType.DMA,
            pltpu.SemaphoreType.DMA,
        ],
        compiler_params=_SC_PARAMS,
    )(functools.partial(_layer_body, n_pad, fg, fs))


def _prep_body(n, fp, x_ref, w1_ref, deg_ref, pre_ref, dinv_ref):
    deg = deg_ref[:n, 0:1] + 1.0
    dinv = lax.rsqrt(deg)
    dinvf = jnp.broadcast_to(dinv, (n, fp))
    pre = jnp.dot(x_ref[...], w1_ref[...], preferred_element_type=jnp.float32)
    pre_ref[...] = pre * dinvf
    dinv_ref[...] = dinvf


def _combine_body(n, fs, s_ref, pre_ref, dinv_ref, b_ref, w_ref, out_ref):
    pre = pre_ref[...]
    s = pre[:, :fs] + s_ref[:n]
    s = jnp.concatenate([s, jnp.zeros_like(pre[:, fs:])], axis=1)
    h = jnp.maximum(s * dinv_ref[...] + b_ref[...], 0.0)
    out_ref[...] = (jnp.dot(h, w_ref[...], preferred_element_type=jnp.float32)
                    * dinv_ref[...])


def _final_body(n, g, f3p, s_ref, pre_ref, dinv_ref, b_ref, batch_ref,
                wf_ref, bf_ref, out_ref):
    h = jnp.maximum(
        (s_ref[:n] + pre_ref[:, :f3p]) * dinv_ref[:, :f3p] + b_ref[...], 0.0)
    onehot = jnp.equal(
        batch_ref[...],
        lax.broadcasted_iota(jnp.int32, (n, g), 1)).astype(jnp.float32)
    h_aug = jnp.concatenate([h, jnp.ones((n, 1), jnp.float32)], axis=1)
    pooled_aug = lax.dot_general(onehot, h_aug, (((0,), (0,)), ((), ())),
                                 preferred_element_type=jnp.float32)
    cnt = pooled_aug[:, -1:]
    pooled = pooled_aug[:, :-1] / jnp.maximum(cnt, 1.0)
    logits = (jnp.dot(pooled, wf_ref[...], preferred_element_type=jnp.float32)
              + bf_ref[...])
    out_ref[...] = jax.nn.softmax(logits, axis=1)


def _tc_call(body, out_shapes):
    return pl.pallas_call(body, out_shape=out_shapes)


def kernel(x, edge_index, edge_weight, batch, W1, b1, W2, b2, W3, b3, Wf, bf):
    n, din = x.shape
    e = edge_index.shape[1]
    f1 = W1.shape[1]
    fp = 128
    f3p = 32
    g = 64
    out_dim = Wf.shape[1]
    n_pad = ((n + 8 * NW - 1) // (8 * NW)) * 8 * NW

    src = edge_index[0]
    dst = edge_index[1]
    w1p = jnp.pad(W1, ((0, 0), (0, fp - f1)))
    w2p = jnp.pad(W2, ((0, fp - f1), (0, fp - f1)))
    w3p = jnp.pad(W3, ((0, fp - f1), (0, fp - W3.shape[1])))
    b1p = jnp.pad(b1, (0, fp - f1)).reshape(1, fp)
    b2p = jnp.pad(b2, (0, fp - f1)).reshape(1, fp)
    b3p = jnp.pad(b3, (0, f3p - b3.shape[0])).reshape(1, f3p)
    wfp = jnp.pad(Wf, ((0, f3p - Wf.shape[0]), (0, 0)))
    fdt = jnp.float32

    msrc, mdl, mew, cnts, deg = _make_part(n_pad, e)(dst, src, edge_weight)
    pre1, dinv = _tc_call(
        functools.partial(_prep_body, n, fp),
        (jax.ShapeDtypeStruct((n, fp), fdt),
         jax.ShapeDtypeStruct((n, fp), fdt)),
    )(x, w1p, deg)

    layer96 = _make_layer(n_pad, fp, f1)
    layer32 = _make_layer(n_pad, fp, f3p)

    s1 = layer96(pre1, msrc, mdl, mew, cnts)
    pre2 = _tc_call(
        functools.partial(_combine_body, n, f1),
        jax.ShapeDtypeStruct((n, fp), fdt),
    )(s1, pre1, dinv, b1p, w2p)

    s2 = layer96(pre2, msrc, mdl, mew, cnts)
    pre3 = _tc_call(
        functools.partial(_combine_body, n, f1),
        jax.ShapeDtypeStruct((n, fp), fdt),
    )(s2, pre2, dinv, b2p, w3p)

    s3 = layer32(pre3, msrc, mdl, mew, cnts)
    out = _tc_call(
        functools.partial(_final_body, n, g, f3p),
        jax.ShapeDtypeStruct((g, out_dim), fdt),
    )(s3, pre3, dinv, b3p, batch.reshape(n, 1), wfp,
      bf.reshape(1, out_dim))
    return out


# speedup vs baseline: 9.6819x; 1.0748x over previous
import functools

import jax
import jax.numpy as jnp
from jax import lax
from jax.experimental import pallas as pl
from jax.experimental.pallas import tpu as pltpu
from jax.experimental.pallas import tpu_sc as plsc

NC = 2
NS = 16
NW = NC * NS
DEGW = 16
CAP = 16384
CSZ = 3200
GC = 128

_SC_PARAMS = pltpu.CompilerParams(needs_layout_passes=False)
_MESH = dict(core_axis_name="c", subcore_axis_name="s")


def _wid():
    return lax.axis_index("c") * NS + lax.axis_index("s")


def _part_body(n_pad, e, dst_hbm, src_hbm, ew_hbm,
               msrc_hbm, mdl_hbm, mew_hbm, cnt_hbm, deg_hbm,
               dstv, srcv, ewv, msrcv, mdlv, mewv, cntv, dega, sem_a, sem_b):
    w = _wid()
    stripe = n_pad // NW
    lo = w * stripe
    z16i = jnp.zeros((16,), jnp.int32)
    z16f = jnp.zeros((16,), jnp.float32)

    def zlist(i, c):
        msrcv[pl.ds(16 * i, 16)] = z16i
        mdlv[pl.ds(16 * i, 16)] = z16i
        mewv[pl.ds(16 * i, 16)] = z16f
        return c

    lax.fori_loop(0, CAP // 16, zlist, 0, unroll=4)

    def zdeg(i, c):
        dega[i] = z16f
        return c

    lax.fori_loop(0, stripe, zdeg, 0, unroll=4)

    nch = e // CSZ
    sems = (sem_a, sem_b)

    def _start(ci, p):
        coff = ci * CSZ
        pltpu.async_copy(dst_hbm.at[pl.ds(coff, CSZ)], dstv.at[p], sems[p])
        pltpu.async_copy(src_hbm.at[pl.ds(coff, CSZ)], srcv.at[p], sems[p])
        pltpu.async_copy(ew_hbm.at[pl.ds(coff, CSZ)], ewv.at[p], sems[p])

    def _wait(p):
        pltpu.make_async_copy(dst_hbm.at[pl.ds(0, CSZ)], dstv.at[p],
                              sems[p]).wait()
        pltpu.make_async_copy(src_hbm.at[pl.ds(0, CSZ)], srcv.at[p],
                              sems[p]).wait()
        pltpu.make_async_copy(ew_hbm.at[pl.ds(0, CSZ)], ewv.at[p],
                              sems[p]).wait()

    def _scan(p, off):
        def vec(i, off2):
            dl = dstv[p, pl.ds(16 * i, 16)] - lo
            m = (dl >= 0) & (dl < stripe)
            sv = srcv[p, pl.ds(16 * i, 16)]
            wv = jnp.abs(ewv[p, pl.ds(16 * i, 16)])
            o = jnp.minimum(off2, CAP - 16)
            plsc.store_compressed(mdlv.at[pl.ds(o, 16)], dl, mask=m)
            plsc.store_compressed(msrcv.at[pl.ds(o, 16)], sv, mask=m)
            plsc.store_compressed(mewv.at[pl.ds(o, 16)], wv, mask=m)
            npop = plsc.all_reduce_population_count(m)
            return off2 + lax.squeeze(lax.slice_in_dim(npop, 0, 1, axis=0), [0])

        return lax.fori_loop(0, CSZ // 16, vec, off, unroll=4)

    _start(0, 0)

    def pair(g2, off):
        ci0 = 2 * g2
        _start(ci0 + 1, 1)
        _wait(0)
        off = _scan(0, off)

        @pl.when(ci0 + 2 < nch)
        def _():
            _start(ci0 + 2, 0)

        _wait(1)
        off = _scan(1, off)
        return off

    off = lax.fori_loop(0, nch // 2, pair, jnp.int32(0))

    cols = lax.iota(jnp.int32, 16)

    def dedge(g2, c):
        for k in range(4):
            gi = jnp.full((16,), 4 * g2 + k, jnp.int32)
            dlv = plsc.load_gather(mdlv, [gi])
            wv = plsc.load_gather(mewv, [gi])
            plsc.addupdate_scatter(dega, [dlv, cols], wv)
        return c

    lax.fori_loop(0, (off + 3) // 4, dedge, 0)

    pltpu.sync_copy(msrcv, msrc_hbm.at[pl.ds(w * CAP, CAP)])
    pltpu.sync_copy(mdlv, mdl_hbm.at[pl.ds(w * CAP, CAP)])
    pltpu.sync_copy(mewv, mew_hbm.at[pl.ds(w * CAP, CAP)])
    cntv[...] = jnp.full((16,), off, jnp.int32)
    pltpu.sync_copy(cntv, cnt_hbm.at[pl.ds(w * 16, 16)])
    pltpu.sync_copy(dega, deg_hbm.at[pl.ds(lo, stripe)])


def _layer_body(n_pad, fg, fs, xw_hbm, msrc_hbm, mdl_hbm, mew_hbm, cnt_hbm,
                out_hbm, msrcv, mdlv, mewv, cntv, rows, accum, sem_a, sem_b):
    w = _wid()
    stripe = n_pad // NW
    lo = w * stripe
    nf = fs // 16
    pltpu.sync_copy(msrc_hbm.at[pl.ds(w * CAP, CAP)], msrcv)
    pltpu.sync_copy(mdl_hbm.at[pl.ds(w * CAP, CAP)], mdlv)
    pltpu.sync_copy(mew_hbm.at[pl.ds(w * CAP, CAP)], mewv)
    pltpu.sync_copy(cnt_hbm.at[pl.ds(w * 16, 16)], cntv)
    z16f = jnp.zeros((16,), jnp.float32)

    def zacc(i, c):
        for j in range(nf):
            accum[i, pl.ds(16 * j, 16)] = z16f
        return c

    lax.fori_loop(0, stripe, zacc, 0, unroll=4)

    cnt = jnp.max(cntv[...])
    trips = (cnt + GC - 1) // GC
    cols = [lax.iota(jnp.int32, 16) + 16 * j for j in range(nf)]
    sems = (sem_a, sem_b)

    def _start(t, p):
        pltpu.async_copy(xw_hbm.at[msrcv.at[pl.ds(t * GC, GC)]], rows.at[p],
                         sems[p])

    def _wait(p):
        pltpu.make_async_copy(xw_hbm.at[pl.ds(0, GC)], rows.at[p],
                              sems[p]).wait()

    def _process(t, p):
        base = t * GC

        def group(k, c2):
            gb = base + 16 * k
            dl16 = mdlv[pl.ds(gb, 16)]
            w16 = mewv[pl.ds(gb, 16)]
            rb = 16 * k
            for j in range(16):
                sj = jnp.full((16,), j, jnp.int32)
                dlv = jnp.take_along_axis(dl16, sj, axis=0,
                                          mode="promise_in_bounds")
                wv = jnp.take_along_axis(w16, sj, axis=0,
                                         mode="promise_in_bounds")
                for q in range(nf):
                    vals = rows[p, rb + j, pl.ds(16 * q, 16)] * wv
                    plsc.addupdate_scatter(accum, [dlv, cols[q]], vals)
            return c2

        lax.fori_loop(0, GC // 16, group, 0)

    @pl.when(trips > 0)
    def _():
        _start(0, 0)

    def pair(g2, c):
        t0 = 2 * g2

        @pl.when(t0 + 1 < trips)
        def _():
            _start(t0 + 1, 1)

        _wait(0)
        _process(t0, 0)

        @pl.when(t0 + 2 < trips)
        def _():
            _start(t0 + 2, 0)

        @pl.when(t0 + 1 < trips)
        def _():
            _wait(1)
            _process(t0 + 1, 1)

        return c

    lax.fori_loop(0, (trips + 1) // 2, pair, 0)
    pltpu.sync_copy(accum, out_hbm.at[pl.ds(lo, stripe)])


def _make_part(n_pad, e):
    mesh = plsc.VectorSubcoreMesh(**_MESH)
    return functools.partial(
        pl.kernel,
        mesh=mesh,
        out_type=(
            jax.ShapeDtypeStruct((NW * CAP,), jnp.int32),
            jax.ShapeDtypeStruct((NW * CAP,), jnp.int32),
            jax.ShapeDtypeStruct((NW * CAP,), jnp.float32),
            jax.ShapeDtypeStruct((NW * 16,), jnp.int32),
            jax.ShapeDtypeStruct((n_pad, DEGW), jnp.float32),
        ),
        scratch_types=[
            pltpu.VMEM((2, CSZ), jnp.int32),
            pltpu.VMEM((2, CSZ), jnp.int32),
            pltpu.VMEM((2, CSZ), jnp.float32),
            pltpu.VMEM((CAP,), jnp.int32),
            pltpu.VMEM((CAP,), jnp.int32),
            pltpu.VMEM((CAP,), jnp.float32),
            pltpu.VMEM((16,), jnp.int32),
            pltpu.VMEM((n_pad // NW, DEGW), jnp.float32),
            pltpu.SemaphoreType.DMA,
            pltpu.SemaphoreType.DMA,
        ],
        compiler_params=_SC_PARAMS,
    )(functools.partial(_part_body, n_pad, e))


def _make_layer(n_pad, fg, fs):
    mesh = plsc.VectorSubcoreMesh(**_MESH)
    return functools.partial(
        pl.kernel,
        mesh=mesh,
        out_type=jax.ShapeDtypeStruct((n_pad, fs), jnp.float32),
        scratch_types=[
            pltpu.VMEM((CAP,), jnp.int32),
            pltpu.VMEM((CAP,), jnp.int32),
            pltpu.VMEM((CAP,), jnp.float32),
            pltpu.VMEM((16,), jnp.int32),
            pltpu.VMEM((2, GC, fg), jnp.float32),
            pltpu.VMEM((n_pad // NW, fs), jnp.float32),
            pltpu.SemaphoreType.DMA,
            pltpu.SemaphoreType.DMA,
        ],
        compiler_params=_SC_PARAMS,
    )(functools.partial(_layer_body, n_pad, fg, fs))


def _prep_body(n, fp, x_ref, w1_ref, deg_ref, pre_ref, dinv_ref):
    deg = deg_ref[:n, 0:1] + 1.0
    dinv = lax.rsqrt(deg)
    dinvf = jnp.broadcast_to(dinv, (n, fp))
    pre = jnp.dot(x_ref[...], w1_ref[...], preferred_element_type=jnp.float32)
    pre_ref[...] = pre * dinvf
    dinv_ref[...] = dinvf


def _combine_body(n, fs, s_ref, pre_ref, dinv_ref, b_ref, w_ref, out_ref):
    pre = pre_ref[...]
    s = pre[:, :fs] + s_ref[:n]
    s = jnp.concatenate([s, jnp.zeros_like(pre[:, fs:])], axis=1)
    h = jnp.maximum(s * dinv_ref[...] + b_ref[...], 0.0)
    out_ref[...] = (jnp.dot(h, w_ref[...], preferred_element_type=jnp.float32)
                    * dinv_ref[...])


def _final_body(n, g, f3p, s_ref, pre_ref, dinv_ref, b_ref, batch_ref,
                wf_ref, bf_ref, out_ref):
    h = jnp.maximum(
        (s_ref[:n] + pre_ref[:, :f3p]) * dinv_ref[:, :f3p] + b_ref[...], 0.0)
    onehot = jnp.equal(
        batch_ref[...],
        lax.broadcasted_iota(jnp.int32, (n, g), 1)).astype(jnp.float32)
    h_aug = jnp.concatenate([h, jnp.ones((n, 1), jnp.float32)], axis=1)
    pooled_aug = lax.dot_general(onehot, h_aug, (((0,), (0,)), ((), ())),
                                 preferred_element_type=jnp.float32)
    cnt = pooled_aug[:, -1:]
    pooled = pooled_aug[:, :-1] / jnp.maximum(cnt, 1.0)
    logits = (jnp.dot(pooled, wf_ref[...], preferred_element_type=jnp.float32)
              + bf_ref[...])
    out_ref[...] = jax.nn.softmax(logits, axis=1)


def _tc_call(body, out_shapes):
    return pl.pallas_call(body, out_shape=out_shapes)


def kernel(x, edge_index, edge_weight, batch, W1, b1, W2, b2, W3, b3, Wf, bf):
    n, din = x.shape
    e = edge_index.shape[1]
    f1 = W1.shape[1]
    fp = 128
    f3p = 32
    g = 64
    out_dim = Wf.shape[1]
    n_pad = ((n + 8 * NW - 1) // (8 * NW)) * 8 * NW

    src = edge_index[0]
    dst = edge_index[1]
    w1p = jnp.pad(W1, ((0, 0), (0, fp - f1)))
    w2p = jnp.pad(W2, ((0, fp - f1), (0, fp - f1)))
    w3p = jnp.pad(W3, ((0, fp - f1), (0, fp - W3.shape[1])))
    b1p = jnp.pad(b1, (0, fp - f1)).reshape(1, fp)
    b2p = jnp.pad(b2, (0, fp - f1)).reshape(1, fp)
    b3p = jnp.pad(b3, (0, f3p - b3.shape[0])).reshape(1, f3p)
    wfp = jnp.pad(Wf, ((0, f3p - Wf.shape[0]), (0, 0)))
    fdt = jnp.float32

    msrc, mdl, mew, cnts, deg = _make_part(n_pad, e)(dst, src, edge_weight)
    pre1, dinv = _tc_call(
        functools.partial(_prep_body, n, fp),
        (jax.ShapeDtypeStruct((n, fp), fdt),
         jax.ShapeDtypeStruct((n, fp), fdt)),
    )(x, w1p, deg)

    layer96 = _make_layer(n_pad, fp, f1)
    layer32 = _make_layer(n_pad, fp, f3p)

    s1 = layer96(pre1, msrc, mdl, mew, cnts)
    pre2 = _tc_call(
        functools.partial(_combine_body, n, f1),
        jax.ShapeDtypeStruct((n, fp), fdt),
    )(s1, pre1, dinv, b1p, w2p)

    s2 = layer96(pre2, msrc, mdl, mew, cnts)
    pre3 = _tc_call(
        functools.partial(_combine_body, n, f1),
        jax.ShapeDtypeStruct((n, fp), fdt),
    )(s2, pre2, dinv, b2p, w3p)

    s3 = layer32(pre3, msrc, mdl, mew, cnts)
    out = _tc_call(
        functools.partial(_final_body, n, g, f3p),
        jax.ShapeDtypeStruct((g, out_dim), fdt),
    )(s3, pre3, dinv, b3p, batch.reshape(n, 1), wfp,
      bf.reshape(1, out_dim))
    return out
